# Initial kernel scaffold; baseline (speedup 1.0000x reference)
import jax
import jax.numpy as jnp
from jax.experimental import pallas as pl


def kernel(x, edge_index, edge_weight, batch, W1, b1, W2, b2):
    raise NotImplementedError("write your pallas kernel here")


# speedup vs baseline: 142.9284x; 142.9284x over previous
import functools

import jax
import jax.numpy as jnp
from jax import lax
from jax.experimental import pallas as pl
from jax.experimental.pallas import tpu as pltpu
from jax.experimental.pallas import tpu_sc as plsc

N = 100000
E = 1600000
G = 64
H = 32
NC, NS, L = 2, 16, 16
NW = NC * NS
NP = 102400
PT = NP // NS
RND = NP // 128
EW = E // NW
C = 2000
NCH = EW // C

_mesh = plsc.VectorSubcoreMesh(core_axis_name="c", subcore_axis_name="s")


def _zero_shared(zbuf, acc, s):
    def zb(i, _):
        zbuf[pl.ds(i * L, L)] = jnp.zeros((L,), jnp.float32)
        return 0
    lax.fori_loop(0, PT // L, zb, 0)
    pltpu.sync_copy(zbuf, acc.at[pl.ds(s * PT, PT)])


@functools.partial(
    pl.kernel,
    out_type=jax.ShapeDtypeStruct((NC * NP,), jnp.float32),
    mesh=_mesh,
    scratch_types=[
        pltpu.VMEM((C,), jnp.int32),
        pltpu.VMEM((C,), jnp.float32),
        pltpu.VMEM((PT,), jnp.float32),
        pltpu.VMEM_SHARED((NP,), jnp.float32),
    ],
    compiler_params=pltpu.CompilerParams(needs_layout_passes=False),
)
def _deg_kernel(dst_hbm, ew_hbm, out_hbm, idx_v, val_v, zbuf, acc):
    c = lax.axis_index("c")
    s = lax.axis_index("s")
    wid = s * NC + c
    _zero_shared(zbuf, acc, s)
    plsc.subcore_barrier()
    base = wid * EW

    def chunk(j, _):
        off = base + j * C
        pltpu.sync_copy(dst_hbm.at[pl.ds(off, C)], idx_v)
        pltpu.sync_copy(ew_hbm.at[pl.ds(off, C)], val_v)
        pltpu.sync_copy(val_v, acc.at[idx_v], add=True)
        return 0

    lax.fori_loop(0, NCH, chunk, 0)
    plsc.subcore_barrier()
    pltpu.sync_copy(acc.at[pl.ds(s * PT, PT)],
                    out_hbm.at[pl.ds(c * NP + s * PT, PT)])


@functools.partial(
    pl.kernel,
    out_type=jax.ShapeDtypeStruct((NC * NP,), jnp.float32),
    mesh=_mesh,
    scratch_types=[
        pltpu.VMEM((N,), jnp.float32),
        pltpu.VMEM((C,), jnp.int32),
        pltpu.VMEM((C,), jnp.int32),
        pltpu.VMEM((C,), jnp.float32),
        pltpu.VMEM((C,), jnp.float32),
        pltpu.VMEM((PT,), jnp.float32),
        pltpu.VMEM_SHARED((NP,), jnp.float32),
    ],
    compiler_params=pltpu.CompilerParams(needs_layout_passes=False),
)
def _edge_kernel(src_hbm, dst_hbm, ew_hbm, tbl_hbm, out_hbm,
                 tbl_v, sidx, didx, w_v, prod, zbuf, acc):
    c = lax.axis_index("c")
    s = lax.axis_index("s")
    wid = s * NC + c
    _zero_shared(zbuf, acc, s)
    pltpu.sync_copy(tbl_hbm, tbl_v)
    plsc.subcore_barrier()
    base = wid * EW

    def chunk(j, _):
        off = base + j * C
        pltpu.sync_copy(src_hbm.at[pl.ds(off, C)], sidx)
        pltpu.sync_copy(dst_hbm.at[pl.ds(off, C)], didx)
        pltpu.sync_copy(ew_hbm.at[pl.ds(off, C)], w_v)

        def inner(i, _):
            s16 = sidx[pl.ds(i * L, L)]
            g16 = plsc.load_gather(tbl_v, [s16])
            prod[pl.ds(i * L, L)] = g16 * w_v[pl.ds(i * L, L)]
            return 0

        lax.fori_loop(0, C // L, inner, 0)
        pltpu.sync_copy(prod, acc.at[didx], add=True)
        return 0

    lax.fori_loop(0, NCH, chunk, 0)
    plsc.subcore_barrier()
    pltpu.sync_copy(acc.at[pl.ds(s * PT, PT)],
                    out_hbm.at[pl.ds(c * NP + s * PT, PT)])


def _dinv_body(degp_ref, x_ref, dinv_ref, p_ref):
    deg = degp_ref[0] + degp_ref[1] + 1.0
    dinv = lax.rsqrt(deg)
    dinv_ref[...] = dinv
    p_ref[...] = dinv * x_ref[...]


_dinv_call = pl.pallas_call(
    _dinv_body,
    out_shape=(jax.ShapeDtypeStruct((RND, 128), jnp.float32),
               jax.ShapeDtypeStruct((RND, 128), jnp.float32)),
)


def _mlp_body(s1p_ref, dinv_ref, p_ref, w1_ref, b1_ref, w2_ref, q_ref):
    dinv = dinv_ref[...]
    a = dinv * (s1p_ref[0] + s1p_ref[1] + p_ref[...])
    t = jnp.zeros_like(a)
    for h in range(H):
        t = t + jnp.maximum(a * w1_ref[0, h] + b1_ref[0, h], 0.0) * w2_ref[0, h]
    q_ref[...] = dinv * t


_mlp_call = pl.pallas_call(
    _mlp_body,
    out_shape=jax.ShapeDtypeStruct((RND, 128), jnp.float32),
)


def _final_body(s2p_ref, dinv_ref, q_ref, batch_ref, b2_ref, out_ref):
    cv = dinv_ref[...] * (s2p_ref[0] + s2p_ref[1] + q_ref[...]) + b2_ref[0, 0]
    b = batch_ref[...]
    sums, cnts = [], []
    for g in range(G):
        m = b == g
        sums.append(jnp.sum(jnp.where(m, cv, 0.0)))
        cnts.append(jnp.sum(jnp.where(m, 1.0, 0.0)))
    out_ref[0, :] = jnp.stack(sums) / jnp.maximum(jnp.stack(cnts), 1.0)


_final_call = pl.pallas_call(
    _final_body,
    out_shape=jax.ShapeDtypeStruct((1, G), jnp.float32),
)


def kernel(x, edge_index, edge_weight, batch, W1, b1, W2, b2):
    src = edge_index[0]
    dst = edge_index[1]
    x_p = jnp.pad(x, (0, NP - N)).reshape(RND, 128)
    batch_p = jnp.pad(batch, (0, NP - N), constant_values=G).reshape(RND, 128)
    w1 = W1.reshape(1, H)
    b1r = b1.reshape(1, H)
    w2 = W2.reshape(1, H)
    b2r = b2.reshape(1, 1)

    degp = _deg_kernel(dst, edge_weight).reshape(NC, RND, 128)
    dinv2, p2 = _dinv_call(degp, x_p)
    s1p = _edge_kernel(src, dst, edge_weight,
                       p2.reshape(NP)[:N]).reshape(NC, RND, 128)
    q2 = _mlp_call(s1p, dinv2, p2, w1, b1r, w2)
    s2p = _edge_kernel(src, dst, edge_weight,
                       q2.reshape(NP)[:N]).reshape(NC, RND, 128)
    out2 = _final_call(s2p, dinv2, q2, batch_p, b2r)
    return out2.reshape(G)


# speedup vs baseline: 254.8975x; 1.7834x over previous
import functools

import jax
import jax.numpy as jnp
from jax import lax
from jax.experimental import pallas as pl
from jax.experimental.pallas import tpu as pltpu
from jax.experimental.pallas import tpu_sc as plsc

N = 100000
E = 1600000
G = 64
H = 32
NC, NS, L = 2, 16, 16
NW = NC * NS
NP = 102400
PT = NP // NS
RND = NP // 128
EW = E // NW
CD = 10000
NCHD = EW // CD
C = 2000
NCH = EW // C

_mesh = plsc.VectorSubcoreMesh(core_axis_name="c", subcore_axis_name="s")


@functools.partial(
    pl.kernel,
    out_type=jax.ShapeDtypeStruct((NC * NP,), jnp.float32),
    mesh=_mesh,
    scratch_types=[
        [pltpu.VMEM((CD,), jnp.int32)] * 3,
        [pltpu.VMEM((CD,), jnp.float32)] * 3,
        pltpu.VMEM((PT,), jnp.float32),
        pltpu.VMEM_SHARED((NP,), jnp.float32),
        [pltpu.SemaphoreType.DMA] * 3,
        [pltpu.SemaphoreType.DMA] * 3,
        [pltpu.SemaphoreType.DMA] * 3,
    ],
    compiler_params=pltpu.CompilerParams(needs_layout_passes=False),
)
def _deg_kernel(dst_hbm, ew_hbm, out_hbm, idx_v, val_v, zbuf, acc,
                lsems_i, lsems_v, ssems):
    c = lax.axis_index("c")
    s = lax.axis_index("s")
    wid = s * NC + c

    def zb(i, _):
        zbuf[pl.ds(i * L, L)] = jnp.zeros((L,), jnp.float32)
        return 0
    lax.fori_loop(0, PT // L, zb, 0)
    pltpu.sync_copy(zbuf, acc.at[pl.ds(s * PT, PT)])
    plsc.subcore_barrier()
    base = wid * EW

    def start_loads(j):
        b = j % 3
        off = base + j * CD
        di = pltpu.async_copy(dst_hbm.at[pl.ds(off, CD)], idx_v[b],
                              lsems_i[b])
        dv = pltpu.async_copy(ew_hbm.at[pl.ds(off, CD)], val_v[b],
                              lsems_v[b])
        return di, dv

    loads = {0: start_loads(0)}
    scats = {}
    for j in range(NCHD):
        b = j % 3
        di, dv = loads.pop(j)
        di.wait()
        dv.wait()
        if j + 1 < NCHD:
            if j - 2 >= 0:
                scats.pop(j - 2).wait()
            loads[j + 1] = start_loads(j + 1)
        scats[j] = pltpu.async_copy(val_v[b], acc.at[idx_v[b]],
                                    ssems[b], add=True)
    for j in sorted(scats):
        scats.pop(j).wait()
    plsc.subcore_barrier()
    pltpu.sync_copy(acc.at[pl.ds(s * PT, PT)],
                    out_hbm.at[pl.ds(c * NP + s * PT, PT)])


@functools.partial(
    pl.kernel,
    out_type=jax.ShapeDtypeStruct((NC * NP,), jnp.float32),
    mesh=_mesh,
    scratch_types=[
        pltpu.VMEM((N,), jnp.float32),
        [pltpu.VMEM((C,), jnp.int32)] * 2,
        [pltpu.VMEM((C,), jnp.int32)] * 3,
        [pltpu.VMEM((C,), jnp.float32)] * 2,
        [pltpu.VMEM((C,), jnp.float32)] * 3,
        pltpu.VMEM_SHARED((NP,), jnp.float32),
        [pltpu.SemaphoreType.DMA] * 2,
        [pltpu.SemaphoreType.DMA] * 3,
        [pltpu.SemaphoreType.DMA] * 2,
        [pltpu.SemaphoreType.DMA] * 3,
        pltpu.SemaphoreType.DMA,
    ],
    compiler_params=pltpu.CompilerParams(needs_layout_passes=False),
)
def _edge_kernel(src_hbm, dst_hbm, ew_hbm, tbl_hbm, out_hbm,
                 tbl_v, sidx, didx, w_v, prod, acc,
                 sem_s, sem_d, sem_w, sem_sc, sem_t):
    c = lax.axis_index("c")
    s = lax.axis_index("s")
    wid = s * NC + c
    tload = pltpu.async_copy(tbl_hbm, tbl_v, sem_t)

    def zb(i, _):
        prod[0][pl.ds(i * L, L)] = jnp.zeros((L,), jnp.float32)
        return 0
    lax.fori_loop(0, C // L, zb, 0)
    for r in range(PT // C):
        pltpu.sync_copy(prod[0], acc.at[pl.ds(s * PT + r * C, C)])
    rem = PT % C
    if rem:
        pltpu.sync_copy(prod[0].at[pl.ds(0, rem)],
                        acc.at[pl.ds(s * PT + (PT // C) * C, rem)])
    tload.wait()
    plsc.subcore_barrier()
    base = wid * EW

    def start_loads(j):
        b2, b3 = j % 2, j % 3
        off = base + j * C
        ds_ = pltpu.async_copy(src_hbm.at[pl.ds(off, C)], sidx[b2],
                               sem_s[b2])
        dd = pltpu.async_copy(dst_hbm.at[pl.ds(off, C)], didx[b3],
                              sem_d[b3])
        dw = pltpu.async_copy(ew_hbm.at[pl.ds(off, C)], w_v[b2],
                              sem_w[b2])
        return ds_, dd, dw

    loads = {0: start_loads(0)}
    scats = {}
    for j in range(NCH):
        b2, b3 = j % 2, j % 3
        for d in loads.pop(j):
            d.wait()
        if j + 1 < NCH:
            if j - 2 >= 0:
                scats.pop(j - 2).wait()
            loads[j + 1] = start_loads(j + 1)

        def inner(i, _, b2=b2, b3=b3):
            s16 = sidx[b2][pl.ds(i * L, L)]
            g16 = plsc.load_gather(tbl_v, [s16])
            prod[b3][pl.ds(i * L, L)] = g16 * w_v[b2][pl.ds(i * L, L)]
            return 0

        lax.fori_loop(0, C // L, inner, 0)
        scats[j] = pltpu.async_copy(prod[b3], acc.at[didx[b3]],
                                    sem_sc[b3], add=True)
    for j in sorted(scats):
        scats.pop(j).wait()
    plsc.subcore_barrier()
    pltpu.sync_copy(acc.at[pl.ds(s * PT, PT)],
                    out_hbm.at[pl.ds(c * NP + s * PT, PT)])


def _dinv_body(degp_ref, x_ref, dinv_ref, p_ref):
    deg = degp_ref[0] + degp_ref[1] + 1.0
    dinv = lax.rsqrt(deg)
    dinv_ref[...] = dinv
    p_ref[...] = dinv * x_ref[...]


_dinv_call = pl.pallas_call(
    _dinv_body,
    out_shape=(jax.ShapeDtypeStruct((RND, 128), jnp.float32),
               jax.ShapeDtypeStruct((RND, 128), jnp.float32)),
)


def _mlp_body(s1p_ref, dinv_ref, p_ref, w1_ref, b1_ref, w2_ref, q_ref):
    dinv = dinv_ref[...]
    a = dinv * (s1p_ref[0] + s1p_ref[1] + p_ref[...])
    t = jnp.zeros_like(a)
    for h in range(H):
        t = t + jnp.maximum(a * w1_ref[0, h] + b1_ref[0, h], 0.0) * w2_ref[0, h]
    q_ref[...] = dinv * t


_mlp_call = pl.pallas_call(
    _mlp_body,
    out_shape=jax.ShapeDtypeStruct((RND, 128), jnp.float32),
)


def _final_body(s2p_ref, dinv_ref, q_ref, batch_ref, b2_ref, out_ref):
    cv = dinv_ref[...] * (s2p_ref[0] + s2p_ref[1] + q_ref[...]) + b2_ref[0, 0]
    b = batch_ref[...]
    sums, cnts = [], []
    for g in range(G):
        m = b == g
        sums.append(jnp.sum(jnp.where(m, cv, 0.0)))
        cnts.append(jnp.sum(jnp.where(m, 1.0, 0.0)))
    out_ref[0, :] = jnp.stack(sums) / jnp.maximum(jnp.stack(cnts), 1.0)


_final_call = pl.pallas_call(
    _final_body,
    out_shape=jax.ShapeDtypeStruct((1, G), jnp.float32),
)


def kernel(x, edge_index, edge_weight, batch, W1, b1, W2, b2):
    src = edge_index[0]
    dst = edge_index[1]
    x_p = jnp.pad(x, (0, NP - N)).reshape(RND, 128)
    batch_p = jnp.pad(batch, (0, NP - N), constant_values=G).reshape(RND, 128)
    w1 = W1.reshape(1, H)
    b1r = b1.reshape(1, H)
    w2 = W2.reshape(1, H)
    b2r = b2.reshape(1, 1)

    degp = _deg_kernel(dst, edge_weight).reshape(NC, RND, 128)
    dinv2, p2 = _dinv_call(degp, x_p)
    s1p = _edge_kernel(src, dst, edge_weight,
                       p2.reshape(NP)[:N]).reshape(NC, RND, 128)
    q2 = _mlp_call(s1p, dinv2, p2, w1, b1r, w2)
    s2p = _edge_kernel(src, dst, edge_weight,
                       q2.reshape(NP)[:N]).reshape(NC, RND, 128)
    out2 = _final_call(s2p, dinv2, q2, batch_p, b2r)
    return out2.reshape(G)
